# baseline (speedup 1.0000x reference)
import functools

import jax
import jax.numpy as jnp
from jax import lax
from jax.experimental import pallas as pl
from jax.experimental.pallas import tpu as pltpu
from jax.experimental.pallas import tpu_sc as plsc

LEN = 1000000
FEAT = 128
SB = 16384
SAMB = 4096

NC = 2
NS = 16
R = 62528
SPH = SAMB // NC
SPW = SAMB // (NC * NS)

_UNROLL = 8
_GS = 8
_GROWS = SPW // _GS

_SC_MESH = plsc.VectorSubcoreMesh(core_axis_name="c", subcore_axis_name="s")
_SC_PARAMS = pltpu.CompilerParams(needs_layout_passes=False)


def _resolve_body(mem_h, idx_h, val_h, samp_h,
                  mrows_h, vrows_h, wts_h,
                  idx_v, chunk_v, samp_v, contrib_v, p_v, pc_v, w_v, blk_v,
                  mrows_v, vrows_v, psum_sh,
                  sem_i, sem_m, sem_v):
    cid = lax.axis_index("c")
    sid = lax.axis_index("s")
    iota = lax.iota(jnp.int32, 16)
    zero16 = jnp.full((16,), 0, jnp.int32)
    onef = jnp.full((16,), 1.0, jnp.float32)
    zerof = jnp.full((16,), 0.0, jnp.float32)

    half = cid * SPH
    soff = sid * SPW
    pltpu.sync_copy(samp_h.at[pl.ds(half, SPH)], samp_v)

    cp_m = [
        pltpu.async_copy(
            mem_h.at[samp_v.at[pl.ds(soff + t * _GROWS, _GROWS)]],
            mrows_v.at[pl.ds(t * _GROWS, _GROWS)], sem_m)
        for t in range(_GS)
    ]
    cp_i = pltpu.async_copy(idx_h, idx_v, sem_i)

    base = sid * R
    cp_i.wait()

    def scan(k0, c):
        kks = [k0 * _UNROLL + u for u in range(_UNROLL)]
        kvs = [idx_v[pl.ds(kk * 16, 16)] for kk in kks]
        rels = [kv - base for kv in kvs]
        ms = [(rel >= 0) & (rel < R) for rel in rels]
        relcs = [jnp.where(m, rel, zero16) for m, rel in zip(ms, rels)]
        kvecs = [kk * 16 + iota for kk in kks]
        for relc, kvec, m in zip(relcs, kvecs, ms):
            plsc.store_scatter(chunk_v, [relc], kvec, mask=m)
        return c
    lax.fori_loop(0, SB // (16 * _UNROLL), scan, 0)

    def resolve(i, c):
        svs = [samp_v[pl.ds((i * 4 + u) * 16, 16)] for u in range(4)]
        rels = [sv - base for sv in svs]
        ms = [(rel >= 0) & (rel < R) for rel in rels]
        relcs = [jnp.where(m, rel, zero16) for m, rel in zip(ms, rels)]
        gs = [plsc.load_gather(chunk_v, [relc]) for relc in relcs]
        for u in range(4):
            enc = (gs[u] & (SB - 1)) + 1
            contrib_v[pl.ds((i * 4 + u) * 16, 16)] = jnp.where(ms[u], enc, zero16)
        return c
    lax.fori_loop(0, SPH // 64, resolve, 0)

    pltpu.sync_copy(contrib_v, psum_sh.at[sid])
    plsc.subcore_barrier()

    pltpu.sync_copy(psum_sh.at[:, pl.ds(soff, SPW)], blk_v)

    def mk(i, c):
        acc = blk_v[0, pl.ds(i * 16, 16)]
        for r in range(1, NS):
            acc = acc + blk_v[r, pl.ds(i * 16, 16)]
        pt = acc - 1
        iv = plsc.load_gather(idx_v, [pt])
        sv = samp_v[pl.ds(soff + i * 16, 16)]
        m = iv == sv
        pc_v[pl.ds(i * 16, 16)] = pt
        w_v[pl.ds(i * 16, 16)] = jnp.where(m, onef, zerof)
        return c
    lax.fori_loop(0, SPW // 16, mk, 0)

    cp_v = [
        pltpu.async_copy(
            val_h.at[pc_v.at[pl.ds(t * _GROWS, _GROWS)]],
            vrows_v.at[pl.ds(t * _GROWS, _GROWS)], sem_v)
        for t in range(_GS)
    ]

    sbase = half + soff
    pltpu.sync_copy(w_v, wts_h.at[pl.ds(sbase, SPW)])
    for cp in cp_m:
        cp.wait()
    pltpu.sync_copy(mrows_v, mrows_h.at[pl.ds(sbase, SPW)])
    for cp in cp_v:
        cp.wait()
    pltpu.sync_copy(vrows_v, vrows_h.at[pl.ds(sbase, SPW)])


_resolve = functools.partial(
    pl.kernel,
    out_type=(
        jax.ShapeDtypeStruct((SAMB, FEAT), jnp.float32),
        jax.ShapeDtypeStruct((SAMB, FEAT), jnp.float32),
        jax.ShapeDtypeStruct((SAMB,), jnp.float32),
    ),
    mesh=_SC_MESH,
    compiler_params=_SC_PARAMS,
    scratch_types=[
        pltpu.VMEM((SB,), jnp.int32),
        pltpu.VMEM((R,), jnp.int32),
        pltpu.VMEM((SPH,), jnp.int32),
        pltpu.VMEM((SPH,), jnp.int32),
        pltpu.VMEM((SPW,), jnp.int32),
        pltpu.VMEM((SPW,), jnp.int32),
        pltpu.VMEM((SPW,), jnp.float32),
        pltpu.VMEM((NS, SPW), jnp.int32),
        pltpu.VMEM((SPW, FEAT), jnp.float32),
        pltpu.VMEM((SPW, FEAT), jnp.float32),
        pltpu.VMEM_SHARED((NS, SPH), jnp.int32),
        pltpu.SemaphoreType.DMA,
        pltpu.SemaphoreType.DMA,
        pltpu.SemaphoreType.DMA,
    ],
)(_resolve_body)


def _select_body(w_ref, m_ref, v_ref, o_ref):
    w = w_ref[...]
    mr = m_ref[...]
    vr = v_ref[...]
    o_ref[...] = mr + w * (vr - mr)


_ROWS_BLK = 512


def kernel(mem, idx, val, sample_idx):
    mrows, vrows, wts = _resolve(mem, idx, val, sample_idx)
    out = pl.pallas_call(
        _select_body,
        grid=(SAMB // _ROWS_BLK,),
        in_specs=[
            pl.BlockSpec((_ROWS_BLK, 1), lambda i: (i, 0)),
            pl.BlockSpec((_ROWS_BLK, FEAT), lambda i: (i, 0)),
            pl.BlockSpec((_ROWS_BLK, FEAT), lambda i: (i, 0)),
        ],
        out_specs=pl.BlockSpec((_ROWS_BLK, FEAT), lambda i: (i, 0)),
        out_shape=jax.ShapeDtypeStruct((SAMB, FEAT), jnp.float32),
    )(wts.reshape(SAMB, 1), mrows, vrows)
    return out


# speedup vs baseline: 1.1397x; 1.1397x over previous
import functools

import jax
import jax.numpy as jnp
from jax import lax
from jax.experimental import pallas as pl
from jax.experimental.pallas import tpu as pltpu
from jax.experimental.pallas import tpu_sc as plsc

LEN = 1000000
FEAT = 128
SB = 16384
SAMB = 4096

NC = 2
NS = 16
R = 62528
SPH = SAMB // NC
SPW = SAMB // (NC * NS)

_UNROLL = 8
_GS = 8
_GROWS = SPW // _GS

_SC_MESH = plsc.VectorSubcoreMesh(core_axis_name="c", subcore_axis_name="s")
_SC_PARAMS = pltpu.CompilerParams(needs_layout_passes=False)


def _resolve_body(mem_h, idx_h, val_h, samp_h,
                  mrows_h, vrows_h, wts_h,
                  idx_v, chunk_v, samp_v, contrib_v, p_v, pc_v, w_v, blk_v,
                  mrows_v, vrows_v, psum_sh,
                  sem_i, sem_m, sem_v):
    cid = lax.axis_index("c")
    sid = lax.axis_index("s")
    iota = lax.iota(jnp.int32, 16)
    zero16 = jnp.full((16,), 0, jnp.int32)
    onef = jnp.full((16,), 1.0, jnp.float32)
    zerof = jnp.full((16,), 0.0, jnp.float32)

    half = cid * SPH
    soff = sid * SPW
    sbase = half + soff
    pltpu.sync_copy(samp_h.at[pl.ds(half, SPH)], samp_v)

    cp_m = [
        pltpu.async_copy(
            mem_h.at[samp_v.at[pl.ds(soff + t * _GROWS, _GROWS)]],
            mrows_v.at[pl.ds(t * _GROWS, _GROWS)], sem_m)
        for t in range(_GS)
    ]
    cp_i = pltpu.async_copy(idx_h, idx_v, sem_i)

    base = sid * R
    cp_i.wait()

    def scan(k0, c):
        kks = [k0 * _UNROLL + u for u in range(_UNROLL)]
        kvs = [idx_v[pl.ds(kk * 16, 16)] for kk in kks]
        rels = [kv - base for kv in kvs]
        ms = [(rel >= 0) & (rel < R) for rel in rels]
        relcs = [jnp.where(m, rel, zero16) for m, rel in zip(ms, rels)]
        kvecs = [kk * 16 + iota for kk in kks]
        for relc, kvec, m in zip(relcs, kvecs, ms):
            plsc.store_scatter(chunk_v, [relc], kvec, mask=m)
        return c
    lax.fori_loop(0, SB // (16 * _UNROLL), scan, 0)

    def resolve(i, c):
        svs = [samp_v[pl.ds((i * 4 + u) * 16, 16)] for u in range(4)]
        rels = [sv - base for sv in svs]
        ms = [(rel >= 0) & (rel < R) for rel in rels]
        relcs = [jnp.where(m, rel, zero16) for m, rel in zip(ms, rels)]
        gs = [plsc.load_gather(chunk_v, [relc]) for relc in relcs]
        for u in range(4):
            enc = (gs[u] & (SB - 1)) + 1
            contrib_v[pl.ds((i * 4 + u) * 16, 16)] = jnp.where(ms[u], enc, zero16)
        return c
    lax.fori_loop(0, SPH // 64, resolve, 0)

    pltpu.sync_copy(contrib_v, psum_sh.at[sid])
    plsc.subcore_barrier()

    pltpu.sync_copy(psum_sh.at[:, pl.ds(soff, SPW)], blk_v)

    anys = []
    for i in range(SPW // 16):
        acc = blk_v[0, pl.ds(i * 16, 16)]
        for r in range(1, NS):
            acc = acc + blk_v[r, pl.ds(i * 16, 16)]
        pt = acc - 1
        iv = plsc.load_gather(idx_v, [pt])
        sv = samp_v[pl.ds(soff + i * 16, 16)]
        m = iv == sv
        pc_v[pl.ds(i * 16, 16)] = pt
        w_v[pl.ds(i * 16, 16)] = jnp.where(m, onef, zerof)
        anys.append(jnp.any(m))

    for g in range(SPW // 16):
        def _fetch(g=g):
            pltpu.sync_copy(
                val_h.at[pc_v.at[pl.ds(g * _GROWS, _GROWS)]],
                vrows_v.at[pl.ds(g * _GROWS, _GROWS)])
        lax.cond(anys[g], _fetch, lambda: None)

    pltpu.sync_copy(w_v, wts_h.at[pl.ds(sbase, SPW)])
    for cp in cp_m:
        cp.wait()
    pltpu.sync_copy(mrows_v, mrows_h.at[pl.ds(sbase, SPW)])
    pltpu.sync_copy(vrows_v, vrows_h.at[pl.ds(sbase, SPW)])


_resolve = functools.partial(
    pl.kernel,
    out_type=(
        jax.ShapeDtypeStruct((SAMB, FEAT), jnp.float32),
        jax.ShapeDtypeStruct((SAMB, FEAT), jnp.float32),
        jax.ShapeDtypeStruct((SAMB,), jnp.float32),
    ),
    mesh=_SC_MESH,
    compiler_params=_SC_PARAMS,
    scratch_types=[
        pltpu.VMEM((SB,), jnp.int32),
        pltpu.VMEM((R,), jnp.int32),
        pltpu.VMEM((SPH,), jnp.int32),
        pltpu.VMEM((SPH,), jnp.int32),
        pltpu.VMEM((SPW,), jnp.int32),
        pltpu.VMEM((SPW,), jnp.int32),
        pltpu.VMEM((SPW,), jnp.float32),
        pltpu.VMEM((NS, SPW), jnp.int32),
        pltpu.VMEM((SPW, FEAT), jnp.float32),
        pltpu.VMEM((SPW, FEAT), jnp.float32),
        pltpu.VMEM_SHARED((NS, SPH), jnp.int32),
        pltpu.SemaphoreType.DMA,
        pltpu.SemaphoreType.DMA,
        pltpu.SemaphoreType.DMA,
    ],
)(_resolve_body)


def _select_body(w_ref, m_ref, v_ref, o_ref):
    w = w_ref[...]
    mr = m_ref[...]
    vr = v_ref[...]
    o_ref[...] = jnp.where(w > 0.5, vr, mr)


_ROWS_BLK = 512


def kernel(mem, idx, val, sample_idx):
    mrows, vrows, wts = _resolve(mem, idx, val, sample_idx)
    out = pl.pallas_call(
        _select_body,
        grid=(SAMB // _ROWS_BLK,),
        in_specs=[
            pl.BlockSpec((_ROWS_BLK, 1), lambda i: (i, 0)),
            pl.BlockSpec((_ROWS_BLK, FEAT), lambda i: (i, 0)),
            pl.BlockSpec((_ROWS_BLK, FEAT), lambda i: (i, 0)),
        ],
        out_specs=pl.BlockSpec((_ROWS_BLK, FEAT), lambda i: (i, 0)),
        out_shape=jax.ShapeDtypeStruct((SAMB, FEAT), jnp.float32),
    )(wts.reshape(SAMB, 1), mrows, vrows)
    return out


# speedup vs baseline: 2.0259x; 1.7776x over previous
import functools

import jax
import jax.numpy as jnp
from jax import lax
from jax.experimental import pallas as pl
from jax.experimental.pallas import tpu as pltpu
from jax.experimental.pallas import tpu_sc as plsc

LEN = 1000000
FEAT = 128
SB = 16384
SAMB = 4096

NC = 2
NS = 16
R = 62528
SPH = SAMB // NC
SPW = SAMB // (NC * NS)

_UNROLL = 8
_GS = 8
_GROWS = SPW // _GS

_SC_MESH = plsc.VectorSubcoreMesh(core_axis_name="c", subcore_axis_name="s")
_SC_PARAMS = pltpu.CompilerParams(needs_layout_passes=False)


def _resolve_body(mem_h, idx_h, val_h, samp_h, out_h,
                  idx_v, chunk_v, samp_v, contrib_v, pc_v, blk_v,
                  mrows_v, vrows_v, psum_sh,
                  sem_i, sem_m):
    cid = lax.axis_index("c")
    sid = lax.axis_index("s")
    iota = lax.iota(jnp.int32, 16)
    zero16 = jnp.full((16,), 0, jnp.int32)

    half = cid * SPH
    soff = sid * SPW
    sbase = half + soff
    pltpu.sync_copy(samp_h.at[pl.ds(half, SPH)], samp_v)

    cp_m = [
        pltpu.async_copy(
            mem_h.at[samp_v.at[pl.ds(soff + t * _GROWS, _GROWS)]],
            mrows_v.at[pl.ds(t * _GROWS, _GROWS)], sem_m)
        for t in range(_GS)
    ]
    cp_i = pltpu.async_copy(idx_h, idx_v, sem_i)

    base = sid * R
    cp_i.wait()

    def scan(k0, c):
        kks = [k0 * _UNROLL + u for u in range(_UNROLL)]
        kvs = [idx_v[pl.ds(kk * 16, 16)] for kk in kks]
        rels = [kv - base for kv in kvs]
        ms = [(rel >= 0) & (rel < R) for rel in rels]
        relcs = [jnp.where(m, rel, zero16) for m, rel in zip(ms, rels)]
        kvecs = [kk * 16 + iota for kk in kks]
        for relc, kvec, m in zip(relcs, kvecs, ms):
            plsc.store_scatter(chunk_v, [relc], kvec, mask=m)
        return c
    lax.fori_loop(0, SB // (16 * _UNROLL), scan, 0)

    def resolve(i, c):
        svs = [samp_v[pl.ds((i * 4 + u) * 16, 16)] for u in range(4)]
        rels = [sv - base for sv in svs]
        ms = [(rel >= 0) & (rel < R) for rel in rels]
        relcs = [jnp.where(m, rel, zero16) for m, rel in zip(ms, rels)]
        gs = [plsc.load_gather(chunk_v, [relc]) for relc in relcs]
        for u in range(4):
            enc = (gs[u] & (SB - 1)) + 1
            contrib_v[pl.ds((i * 4 + u) * 16, 16)] = jnp.where(ms[u], enc, zero16)
        return c
    lax.fori_loop(0, SPH // 64, resolve, 0)

    pltpu.sync_copy(contrib_v, psum_sh.at[sid])
    plsc.subcore_barrier()

    pltpu.sync_copy(psum_sh.at[:, pl.ds(soff, SPW)], blk_v)

    match_vecs = []
    anys = []
    for i in range(SPW // 16):
        acc = blk_v[0, pl.ds(i * 16, 16)]
        for r in range(1, NS):
            acc = acc + blk_v[r, pl.ds(i * 16, 16)]
        pt = acc - 1
        iv = plsc.load_gather(idx_v, [pt])
        sv = samp_v[pl.ds(soff + i * 16, 16)]
        m = iv == sv
        pc_v[pl.ds(i * 16, 16)] = pt
        match_vecs.append(m)
        anys.append(jnp.any(m))

    for cp in cp_m:
        cp.wait()

    for g in range(SPW // 16):
        m = match_vecs[g]

        def _fetch_merge(g=g, m=m):
            pltpu.sync_copy(
                val_h.at[pc_v.at[pl.ds(g * _GROWS, _GROWS)]],
                vrows_v.at[pl.ds(g * _GROWS, _GROWS)])
            for r in range(_GROWS):
                def _row(gr=g * _GROWS + r):
                    for c in range(FEAT // 16):
                        mrows_v[gr, pl.ds(c * 16, 16)] = (
                            vrows_v[gr, pl.ds(c * 16, 16)])
                lax.cond(jnp.any(m & (iota == r)), _row, lambda: None)

        lax.cond(anys[g], _fetch_merge, lambda: None)

    pltpu.sync_copy(mrows_v, out_h.at[pl.ds(sbase, SPW)])


_resolve = functools.partial(
    pl.kernel,
    out_type=jax.ShapeDtypeStruct((SAMB, FEAT), jnp.float32),
    mesh=_SC_MESH,
    compiler_params=_SC_PARAMS,
    scratch_types=[
        pltpu.VMEM((SB,), jnp.int32),
        pltpu.VMEM((R,), jnp.int32),
        pltpu.VMEM((SPH,), jnp.int32),
        pltpu.VMEM((SPH,), jnp.int32),
        pltpu.VMEM((SPW,), jnp.int32),
        pltpu.VMEM((NS, SPW), jnp.int32),
        pltpu.VMEM((SPW, FEAT), jnp.float32),
        pltpu.VMEM((SPW, FEAT), jnp.float32),
        pltpu.VMEM_SHARED((NS, SPH), jnp.int32),
        pltpu.SemaphoreType.DMA,
        pltpu.SemaphoreType.DMA,
    ],
)(_resolve_body)


def kernel(mem, idx, val, sample_idx):
    return _resolve(mem, idx, val, sample_idx)


# speedup vs baseline: 2.1097x; 1.0413x over previous
import functools

import jax
import jax.numpy as jnp
from jax import lax
from jax.experimental import pallas as pl
from jax.experimental.pallas import tpu as pltpu
from jax.experimental.pallas import tpu_sc as plsc

LEN = 1000000
FEAT = 128
SB = 16384
SAMB = 4096

NC = 2
NS = 16
R = 62528
SPH = SAMB // NC
SPW = SAMB // (NC * NS)

_UNROLL = 8
_GS = 16
_GROWS = SPW // _GS
_MG = 16

_SC_MESH = plsc.VectorSubcoreMesh(core_axis_name="c", subcore_axis_name="s")
_SC_PARAMS = pltpu.CompilerParams(needs_layout_passes=False)


def _resolve_body(mem_h, idx_h, val_h, samp_h, out_h,
                  idx_v, chunk_v, samp_v, contrib_v, pc_v, blk_v,
                  mrows_v, vrows_v, psum_sh,
                  sem_i, sem_m):
    cid = lax.axis_index("c")
    sid = lax.axis_index("s")
    iota = lax.iota(jnp.int32, 16)
    zero16 = jnp.full((16,), 0, jnp.int32)

    half = cid * SPH
    soff = sid * SPW
    sbase = half + soff
    pltpu.sync_copy(samp_h.at[pl.ds(half, SPH)], samp_v)

    cp_m = [
        pltpu.async_copy(
            mem_h.at[samp_v.at[pl.ds(soff + t * _GROWS, _GROWS)]],
            mrows_v.at[pl.ds(t * _GROWS, _GROWS)], sem_m)
        for t in range(_GS)
    ]
    cp_i = pltpu.async_copy(idx_h, idx_v, sem_i)

    base = sid * R
    cp_i.wait()

    def scan(k0, c):
        kks = [k0 * _UNROLL + u for u in range(_UNROLL)]
        kvs = [idx_v[pl.ds(kk * 16, 16)] for kk in kks]
        rels = [kv - base for kv in kvs]
        ms = [(rel >= 0) & (rel < R) for rel in rels]
        relcs = [jnp.where(m, rel, zero16) for m, rel in zip(ms, rels)]
        kvecs = [kk * 16 + iota for kk in kks]
        for relc, kvec, m in zip(relcs, kvecs, ms):
            plsc.store_scatter(chunk_v, [relc], kvec, mask=m)
        return c
    lax.fori_loop(0, SB // (16 * _UNROLL), scan, 0)

    def resolve(i, c):
        svs = [samp_v[pl.ds((i * 4 + u) * 16, 16)] for u in range(4)]
        rels = [sv - base for sv in svs]
        ms = [(rel >= 0) & (rel < R) for rel in rels]
        relcs = [jnp.where(m, rel, zero16) for m, rel in zip(ms, rels)]
        gs = [plsc.load_gather(chunk_v, [relc]) for relc in relcs]
        for u in range(4):
            enc = (gs[u] & (SB - 1)) + 1
            contrib_v[pl.ds((i * 4 + u) * 16, 16)] = jnp.where(ms[u], enc, zero16)
        return c
    lax.fori_loop(0, SPH // 64, resolve, 0)

    pltpu.sync_copy(contrib_v, psum_sh.at[sid])
    plsc.subcore_barrier()

    pltpu.sync_copy(psum_sh.at[:, pl.ds(soff, SPW)], blk_v)

    match_vecs = []
    anys = []
    for i in range(SPW // 16):
        acc = blk_v[0, pl.ds(i * 16, 16)]
        for r in range(1, NS):
            acc = acc + blk_v[r, pl.ds(i * 16, 16)]
        pt = acc - 1
        iv = plsc.load_gather(idx_v, [pt])
        sv = samp_v[pl.ds(soff + i * 16, 16)]
        m = iv == sv
        pc_v[pl.ds(i * 16, 16)] = pt
        match_vecs.append(m)
        anys.append(jnp.any(m))

    for cp in cp_m:
        cp.wait()

    for g in range(SPW // _MG):
        m = match_vecs[g]

        def _fetch_merge(g=g, m=m):
            pltpu.sync_copy(
                val_h.at[pc_v.at[pl.ds(g * _MG, _MG)]],
                vrows_v.at[pl.ds(g * _MG, _MG)])
            for r in range(_MG):
                def _row(gr=g * _MG + r):
                    for c in range(FEAT // 16):
                        mrows_v[gr, pl.ds(c * 16, 16)] = (
                            vrows_v[gr, pl.ds(c * 16, 16)])
                lax.cond(jnp.any(m & (iota == r)), _row, lambda: None)

        lax.cond(anys[g], _fetch_merge, lambda: None)

    pltpu.sync_copy(mrows_v, out_h.at[pl.ds(sbase, SPW)])


_resolve = functools.partial(
    pl.kernel,
    out_type=jax.ShapeDtypeStruct((SAMB, FEAT), jnp.float32),
    mesh=_SC_MESH,
    compiler_params=_SC_PARAMS,
    scratch_types=[
        pltpu.VMEM((SB,), jnp.int32),
        pltpu.VMEM((R,), jnp.int32),
        pltpu.VMEM((SPH,), jnp.int32),
        pltpu.VMEM((SPH,), jnp.int32),
        pltpu.VMEM((SPW,), jnp.int32),
        pltpu.VMEM((NS, SPW), jnp.int32),
        pltpu.VMEM((SPW, FEAT), jnp.float32),
        pltpu.VMEM((SPW, FEAT), jnp.float32),
        pltpu.VMEM_SHARED((NS, SPH), jnp.int32),
        pltpu.SemaphoreType.DMA,
        pltpu.SemaphoreType.DMA,
    ],
)(_resolve_body)


def kernel(mem, idx, val, sample_idx):
    return _resolve(mem, idx, val, sample_idx)
